# baseline (speedup 1.0000x reference)
import functools

import jax
import jax.numpy as jnp
from jax import lax
from jax.experimental import pallas as pl
from jax.experimental.pallas import tpu as pltpu
from jax.experimental.pallas import tpu_sc as plsc

B = 16384
D = 16
NROW = 1000000
NC = 2
NS = 16
NW = NC * NS
CH = B // NW
WV = 8
NWAVE = CH // WV

_mesh = plsc.VectorSubcoreMesh(core_axis_name="c", subcore_axis_name="s")


@functools.partial(
    pl.kernel,
    out_type=jax.ShapeDtypeStruct((B,), jnp.float32),
    mesh=_mesh,
    compiler_params=pltpu.CompilerParams(
        needs_layout_passes=False, use_tc_tiling_on_sc=True),
    scratch_types=dict(
        u_v=pltpu.VMEM((CH + D,), jnp.int32),
        i_v=pltpu.VMEM((CH + D,), jnp.int32),
        blk=pltpu.VMEM((2 * D, D, 128), jnp.float32),
        vu_rows=pltpu.VMEM((D * D,), jnp.float32),
        prod=pltpu.VMEM((D * D,), jnp.float32),
        g_ubm=pltpu.VMEM((CH,), jnp.float32),
        g_ubl=pltpu.VMEM((CH,), jnp.float32),
        g_ibm=pltpu.VMEM((CH,), jnp.float32),
        g_ibl=pltpu.VMEM((CH,), jnp.float32),
        evu_r=pltpu.VMEM((CH * D,), jnp.float32),
        evi_r=pltpu.VMEM((CH * D,), jnp.float32),
        l_ebu=pltpu.VMEM((CH,), jnp.float32),
        l_ebi=pltpu.VMEM((CH,), jnp.float32),
        l_glob=pltpu.VMEM((D,), jnp.float32),
        out_v=pltpu.VMEM((CH,), jnp.float32),
        sem=pltpu.SemaphoreType.DMA,
        semu=pltpu.SemaphoreType.DMA,
        semi=pltpu.SemaphoreType.DMA,
    ),
)
def _vmf_sc(u, i, ubm, ubl, uvm, uvl, ibm, ibl, ivm, ivl, glob,
            ebu, evu, ebi, evi, out,
            u_v, i_v, blk, vu_rows, prod,
            g_ubm, g_ubl, g_ibm, g_ibl,
            evu_r, evi_r, l_ebu, l_ebi, l_glob, out_v, sem, semu, semi):
  wid = lax.axis_index("s") * NC + lax.axis_index("c")
  base = wid * CH

  pltpu.sync_copy(u.at[pl.ds(base, CH)], u_v.at[pl.ds(0, CH)])
  pltpu.sync_copy(i.at[pl.ds(base, CH)], i_v.at[pl.ds(0, CH)])

  cps = [
      pltpu.async_copy(ebu.at[pl.ds(base, CH)], l_ebu, sem),
      pltpu.async_copy(ebi.at[pl.ds(base, CH)], l_ebi, sem),
      pltpu.async_copy(evu.at[pl.ds(base * D, CH * D)], evu_r, sem),
      pltpu.async_copy(evi.at[pl.ds(base * D, CH * D)], evi_r, sem),
      pltpu.async_copy(glob, l_glob, sem),
      pltpu.async_copy(ubm.at[u_v.at[pl.ds(0, CH)]], g_ubm, sem),
      pltpu.async_copy(ubl.at[u_v.at[pl.ds(0, CH)]], g_ubl, sem),
      pltpu.async_copy(ibm.at[i_v.at[pl.ds(0, CH)]], g_ibm, sem),
      pltpu.async_copy(ibl.at[i_v.at[pl.ds(0, CH)]], g_ibl, sem),
  ]

  lane = lax.broadcasted_iota(jnp.int32, (D,), 0)

  def fire_wave(mu_t, lv_t, idx_ref, w, boff, dsem):
    idx8 = idx_ref[pl.ds(w * WV, D)]
    b8 = idx8 - jnp.bitwise_and(idx8, 127)
    for l in range(WV):
      b = pl.multiple_of(b8[l], 128)
      sl = pl.ds(b, 128)
      pltpu.async_copy(mu_t.at[:, sl], blk.at[boff + 2 * l], dsem)
      pltpu.async_copy(lv_t.at[:, sl], blk.at[boff + 2 * l + 1], dsem)

  def drain_wave(boff, dsem):
    for l in range(2 * WV):
      pltpu.make_async_copy(
          uvm.at[:, pl.ds(0, 128)], blk.at[boff + l], dsem).wait()

  fire_wave(uvm, uvl, u_v, 0, 0, semu)
  fire_wave(ivm, ivl, i_v, 0, D, semi)

  for cp in cps:
    cp.wait()
  gv = l_glob[...]

  def wave_body(w, carry):
    r0 = w * WV
    wn = lax.rem(w + 1, NWAVE)
    u16 = u_v[pl.ds(r0, D)]
    i16 = i_v[pl.ds(r0, D)]
    uc16 = jnp.bitwise_and(u16, 127)
    ic16 = jnp.bitwise_and(i16, 127)

    drain_wave(0, semu)
    for l in range(WV):
      col = jnp.full((D,), uc16[l], jnp.int32)
      mu = plsc.load_gather(blk.at[2 * l], [lane, col])
      lv = plsc.load_gather(blk.at[2 * l + 1], [lane, col])
      ev = evu_r[pl.ds((r0 + l) * D, D)]
      vu_rows[pl.ds(l * D, D)] = mu + jnp.exp(0.5 * lv) * ev
    fire_wave(uvm, uvl, u_v, wn, 0, semu)

    half = lax.rem(w, 2) * WV
    drain_wave(D, semi)
    for l in range(WV):
      col = jnp.full((D,), ic16[l], jnp.int32)
      mu = plsc.load_gather(blk.at[D + 2 * l], [lane, col])
      lv = plsc.load_gather(blk.at[D + 2 * l + 1], [lane, col])
      ev = evi_r[pl.ds((r0 + l) * D, D)]
      vi = mu + jnp.exp(0.5 * lv) * ev
      p = vu_rows[pl.ds(l * D, D)] * vi
      plsc.store_scatter(prod, [lane * D + half + l], p)
    fire_wave(ivm, ivl, i_v, wn, D, semi)

    @pl.when(lax.rem(w, 2) == 1)
    def _tail():
      acc = prod[pl.ds(0, D)]
      for c in range(1, D):
        acc = acc + prod[pl.ds(c * D, D)]
      g0 = (w - 1) * WV
      sl = pl.ds(g0, D)
      bu = g_ubm[sl] + jnp.exp(0.5 * g_ubl[sl]) * l_ebu[sl]
      bi = g_ibm[sl] + jnp.exp(0.5 * g_ibl[sl]) * l_ebi[sl]
      out_v[sl] = bu + bi + gv + acc

    return carry

  lax.fori_loop(0, NWAVE, wave_body, 0)

  drain_wave(0, semu)
  drain_wave(D, semi)

  pltpu.sync_copy(out_v, out.at[pl.ds(base, CH)])


def kernel(u, i, user_bias_mu, user_bias_lv, user_vect_mu, user_vect_lv,
           item_bias_mu, item_bias_lv, item_vect_mu, item_vect_lv,
           glob_bias, eps_bu, eps_vu, eps_bi, eps_vi):
  return _vmf_sc(
      u, i,
      user_bias_mu.reshape(-1), user_bias_lv.reshape(-1),
      user_vect_mu.T, user_vect_lv.T,
      item_bias_mu.reshape(-1), item_bias_lv.reshape(-1),
      item_vect_mu.T, item_vect_lv.T,
      jnp.broadcast_to(glob_bias.reshape(1), (D,)),
      eps_bu, eps_vu.reshape(-1), eps_bi, eps_vi.reshape(-1))


# speedup vs baseline: 1.0071x; 1.0071x over previous
import functools

import jax
import jax.numpy as jnp
from jax import lax
from jax.experimental import pallas as pl
from jax.experimental.pallas import tpu as pltpu
from jax.experimental.pallas import tpu_sc as plsc

B = 16384
D = 16
NROW = 1000000
NC = 2
NS = 16
NW = NC * NS
CH = B // NW
NCK = CH // D

_mesh = plsc.VectorSubcoreMesh(core_axis_name="c", subcore_axis_name="s")


@functools.partial(
    pl.kernel,
    out_type=jax.ShapeDtypeStruct((B,), jnp.float32),
    mesh=_mesh,
    compiler_params=pltpu.CompilerParams(
        needs_layout_passes=False, use_tc_tiling_on_sc=True),
    scratch_types=dict(
        u_v=pltpu.VMEM((CH,), jnp.int32),
        i_v=pltpu.VMEM((CH,), jnp.int32),
        blk=pltpu.VMEM((2 * D, D, 128), jnp.float32),
        vu_rows=pltpu.VMEM((D * D,), jnp.float32),
        prod=pltpu.VMEM((D * D,), jnp.float32),
        g_ubm=pltpu.VMEM((CH,), jnp.float32),
        g_ubl=pltpu.VMEM((CH,), jnp.float32),
        g_ibm=pltpu.VMEM((CH,), jnp.float32),
        g_ibl=pltpu.VMEM((CH,), jnp.float32),
        l_evu=pltpu.VMEM((D, CH), jnp.float32),
        l_evi=pltpu.VMEM((D, CH), jnp.float32),
        l_ebu=pltpu.VMEM((CH,), jnp.float32),
        l_ebi=pltpu.VMEM((CH,), jnp.float32),
        l_glob=pltpu.VMEM((D,), jnp.float32),
        out_v=pltpu.VMEM((CH,), jnp.float32),
        sem=pltpu.SemaphoreType.DMA,
        gsem=pltpu.SemaphoreType.DMA,
    ),
)
def _vmf_sc(u, i, ubm, ubl, uvm, uvl, ibm, ibl, ivm, ivl, glob,
            ebu, evu, ebi, evi, out,
            u_v, i_v, blk, vu_rows, prod,
            g_ubm, g_ubl, g_ibm, g_ibl,
            l_evu, l_evi, l_ebu, l_ebi, l_glob, out_v, sem, gsem):
  wid = lax.axis_index("s") * NC + lax.axis_index("c")
  base = wid * CH

  pltpu.sync_copy(u.at[pl.ds(base, CH)], u_v)
  pltpu.sync_copy(i.at[pl.ds(base, CH)], i_v)

  cps = [
      pltpu.async_copy(ebu.at[pl.ds(base, CH)], l_ebu, sem),
      pltpu.async_copy(ebi.at[pl.ds(base, CH)], l_ebi, sem),
      pltpu.async_copy(evu.at[:, pl.ds(base, CH)], l_evu, sem),
      pltpu.async_copy(evi.at[:, pl.ds(base, CH)], l_evi, sem),
      pltpu.async_copy(glob, l_glob, sem),
      pltpu.async_copy(ubm.at[u_v], g_ubm, sem),
      pltpu.async_copy(ubl.at[u_v], g_ubl, sem),
      pltpu.async_copy(ibm.at[i_v], g_ibm, sem),
      pltpu.async_copy(ibl.at[i_v], g_ibl, sem),
  ]
  for cp in cps:
    cp.wait()

  gv = l_glob[...]
  lane = lax.broadcasted_iota(jnp.int32, (D,), 0)

  def fetch_side(mu_t, lv_t, base16):
    grp = []
    for l in range(D):
      b = pl.multiple_of(base16[l], 128)
      sl = pl.ds(b, 128)
      grp += [
          pltpu.async_copy(mu_t.at[:, sl], blk.at[2 * l], gsem),
          pltpu.async_copy(lv_t.at[:, sl], blk.at[2 * l + 1], gsem),
      ]
    return grp

  def group_body(k, carry):
    r0 = k * D
    u16 = u_v[pl.ds(r0, D)]
    i16 = i_v[pl.ds(r0, D)]
    ub16 = u16 - jnp.bitwise_and(u16, 127)
    ib16 = i16 - jnp.bitwise_and(i16, 127)
    uc16 = jnp.bitwise_and(u16, 127)
    ic16 = jnp.bitwise_and(i16, 127)

    for cp in fetch_side(uvm, uvl, ub16):
      cp.wait()
    for l in range(D):
      col = jnp.full((D,), uc16[l], jnp.int32)
      mu = plsc.load_gather(blk.at[2 * l], [lane, col])
      lv = plsc.load_gather(blk.at[2 * l + 1], [lane, col])
      ev = plsc.load_gather(l_evu, [lane, jnp.full((D,), r0 + l, jnp.int32)])
      vu_rows[pl.ds(l * D, D)] = mu + jnp.exp(0.5 * lv) * ev

    for cp in fetch_side(ivm, ivl, ib16):
      cp.wait()
    for l in range(D):
      col = jnp.full((D,), ic16[l], jnp.int32)
      mu = plsc.load_gather(blk.at[2 * l], [lane, col])
      lv = plsc.load_gather(blk.at[2 * l + 1], [lane, col])
      ev = plsc.load_gather(l_evi, [lane, jnp.full((D,), r0 + l, jnp.int32)])
      vi = mu + jnp.exp(0.5 * lv) * ev
      p = vu_rows[pl.ds(l * D, D)] * vi
      plsc.store_scatter(prod, [lane * D + l], p)

    acc = prod[pl.ds(0, D)]
    for c in range(1, D):
      acc = acc + prod[pl.ds(c * D, D)]
    sl = pl.ds(r0, D)
    bu = g_ubm[sl] + jnp.exp(0.5 * g_ubl[sl]) * l_ebu[sl]
    bi = g_ibm[sl] + jnp.exp(0.5 * g_ibl[sl]) * l_ebi[sl]
    out_v[sl] = bu + bi + gv + acc
    return carry

  lax.fori_loop(0, NCK, group_body, 0)

  pltpu.sync_copy(out_v, out.at[pl.ds(base, CH)])


def kernel(u, i, user_bias_mu, user_bias_lv, user_vect_mu, user_vect_lv,
           item_bias_mu, item_bias_lv, item_vect_mu, item_vect_lv,
           glob_bias, eps_bu, eps_vu, eps_bi, eps_vi):
  return _vmf_sc(
      u, i,
      user_bias_mu.reshape(-1), user_bias_lv.reshape(-1),
      user_vect_mu.T, user_vect_lv.T,
      item_bias_mu.reshape(-1), item_bias_lv.reshape(-1),
      item_vect_mu.T, item_vect_lv.T,
      jnp.broadcast_to(glob_bias.reshape(1), (D,)),
      eps_bu, eps_vu.T, eps_bi, eps_vi.T)
